# Initial kernel scaffold; baseline (speedup 1.0000x reference)
import jax
import jax.numpy as jnp
from jax.experimental import pallas as pl


def kernel(ufeat, ifeat, edge_index):
    raise NotImplementedError("write your pallas kernel here")


# speedup vs baseline: 3.4669x; 3.4669x over previous
import dataclasses
import functools

import jax
import jax.numpy as jnp
from jax import lax
from jax.experimental import pallas as pl
from jax.experimental.pallas import tpu as pltpu
from jax.experimental.pallas import tpu_sc as plsc

D = 128
LANES = 16
NUM_CORES = 2
NUM_SUBCORES = 16
NW = NUM_CORES * NUM_SUBCORES


def _dot_kernel(E, W, ufeat_hbm, ifeat_hbm, src_hbm, dst_hbm, out_hbm,
                idx_u, idx_v, u_rows, v_rows, red, out_v, sem_u, sem_v):
    per_w = E // NW
    n_chunks = per_w // W
    wid = lax.axis_index("s") * NUM_CORES + lax.axis_index("c")
    col0 = lax.iota(jnp.int32, LANES) * LANES

    @pl.loop(0, n_chunks)
    def _(c):
        base = wid * per_w + c * W
        pltpu.sync_copy(src_hbm.at[pl.ds(base, W)], idx_u)
        pltpu.sync_copy(dst_hbm.at[pl.ds(base, W)], idx_v)
        cp_u = pltpu.async_copy(ufeat_hbm.at[idx_u], u_rows, sem_u)
        cp_v = pltpu.async_copy(ifeat_hbm.at[idx_v], v_rows, sem_v)
        cp_u.wait()
        cp_v.wait()

        @pl.loop(0, W // LANES)
        def _(g):
            for e in range(LANES):
                row = g * LANES + e
                acc = u_rows[row, pl.ds(0, LANES)] * v_rows[row, pl.ds(0, LANES)]
                for k in range(1, D // LANES):
                    acc = acc + (u_rows[row, pl.ds(k * LANES, LANES)]
                                 * v_rows[row, pl.ds(k * LANES, LANES)])
                red[pl.ds(e * LANES, LANES)] = acc
            tot = plsc.load_gather(red, [col0])
            for j in range(1, LANES):
                tot = tot + plsc.load_gather(red, [col0 + j])
            out_v[pl.ds(g * LANES, LANES)] = tot

        pltpu.sync_copy(out_v, out_hbm.at[pl.ds(base, W)])


def _build_sc_call(E, W):
    mesh = plsc.VectorSubcoreMesh(core_axis_name="c", subcore_axis_name="s")
    cp = pltpu.CompilerParams()
    if "needs_layout_passes" in pltpu.CompilerParams.__dataclass_fields__:
        cp = dataclasses.replace(cp, needs_layout_passes=False)
    return pl.kernel(
        functools.partial(_dot_kernel, E, W),
        out_type=jax.ShapeDtypeStruct((E,), jnp.float32),
        mesh=mesh,
        scratch_types=[
            pltpu.VMEM((W,), jnp.int32),
            pltpu.VMEM((W,), jnp.int32),
            pltpu.VMEM((W, D), jnp.float32),
            pltpu.VMEM((W, D), jnp.float32),
            pltpu.VMEM((LANES * LANES,), jnp.float32),
            pltpu.VMEM((W,), jnp.float32),
            pltpu.SemaphoreType.DMA,
            pltpu.SemaphoreType.DMA,
        ],
        compiler_params=cp,
    )


@jax.jit
def kernel(ufeat, ifeat, edge_index):
    E = edge_index.shape[1]
    src = edge_index[0].astype(jnp.int32)
    dst = edge_index[1].astype(jnp.int32)
    sr = _build_sc_call(E, 80)(ufeat, ifeat, src, dst)
    return sr.reshape(E, 1)


# speedup vs baseline: 7.7155x; 2.2255x over previous
import dataclasses
import functools

import jax
import jax.numpy as jnp
from jax import lax
from jax.experimental import pallas as pl
from jax.experimental.pallas import tpu as pltpu
from jax.experimental.pallas import tpu_sc as plsc

D = 128
LANES = 16
NUM_CORES = 2
NUM_SUBCORES = 16
NW = NUM_CORES * NUM_SUBCORES


def _dot_kernel(E, W, ufeat_hbm, ifeat_hbm, src_hbm, dst_hbm, out_hbm,
                idx_u, idx_v, u_a, v_a, u_b, v_b, red, out_all,
                sem_ua, sem_va, sem_ub, sem_vb):
    per_w = E // NW
    n_chunks = per_w // W
    wid = lax.axis_index("s") * NUM_CORES + lax.axis_index("c")
    base_w = wid * per_w
    col0 = lax.iota(jnp.int32, LANES) * LANES

    pltpu.sync_copy(src_hbm.at[pl.ds(base_w, per_w)], idx_u)
    pltpu.sync_copy(dst_hbm.at[pl.ds(base_w, per_w)], idx_v)

    def gather(c, u_buf, v_buf, sem_u, sem_v):
        pltpu.make_async_copy(
            ufeat_hbm.at[idx_u.at[pl.ds(c * W, W)]], u_buf, sem_u).start()
        pltpu.make_async_copy(
            ifeat_hbm.at[idx_v.at[pl.ds(c * W, W)]], v_buf, sem_v).start()

    def wait(c, u_buf, v_buf, sem_u, sem_v):
        pltpu.make_async_copy(
            ufeat_hbm.at[idx_u.at[pl.ds(c * W, W)]], u_buf, sem_u).wait()
        pltpu.make_async_copy(
            ifeat_hbm.at[idx_v.at[pl.ds(c * W, W)]], v_buf, sem_v).wait()

    def compute(c, u_buf, v_buf):
        @pl.loop(0, W // LANES)
        def _(g):
            for e in range(LANES):
                row = g * LANES + e
                acc = u_buf[row, pl.ds(0, LANES)] * v_buf[row, pl.ds(0, LANES)]
                for k in range(1, D // LANES):
                    acc = acc + (u_buf[row, pl.ds(k * LANES, LANES)]
                                 * v_buf[row, pl.ds(k * LANES, LANES)])
                red[pl.ds(e * LANES, LANES)] = acc
            tot = plsc.load_gather(red, [col0])
            for j in range(1, LANES):
                tot = tot + plsc.load_gather(red, [col0 + j])
            out_all[pl.ds(c * W + g * LANES, LANES)] = tot

    gather(0, u_a, v_a, sem_ua, sem_va)
    gather(1, u_b, v_b, sem_ub, sem_vb)
    wait(0, u_a, v_a, sem_ua, sem_va)
    compute(0, u_a, v_a)
    gather(2, u_a, v_a, sem_ua, sem_va)

    @pl.loop(1, n_chunks - 2, step=2)
    def _(c):
        wait(c, u_b, v_b, sem_ub, sem_vb)
        compute(c, u_b, v_b)
        gather(c + 2, u_b, v_b, sem_ub, sem_vb)
        wait(c + 1, u_a, v_a, sem_ua, sem_va)
        compute(c + 1, u_a, v_a)
        gather(c + 3, u_a, v_a, sem_ua, sem_va)

    wait(n_chunks - 2, u_b, v_b, sem_ub, sem_vb)
    compute(n_chunks - 2, u_b, v_b)
    wait(n_chunks - 1, u_a, v_a, sem_ua, sem_va)
    compute(n_chunks - 1, u_a, v_a)

    pltpu.sync_copy(out_all, out_hbm.at[pl.ds(base_w, per_w)])


def _build_sc_call(E, W):
    per_w = E // NW
    mesh = plsc.VectorSubcoreMesh(core_axis_name="c", subcore_axis_name="s")
    cp = pltpu.CompilerParams()
    if "needs_layout_passes" in pltpu.CompilerParams.__dataclass_fields__:
        cp = dataclasses.replace(cp, needs_layout_passes=False)
    return pl.kernel(
        functools.partial(_dot_kernel, E, W),
        out_type=jax.ShapeDtypeStruct((E,), jnp.float32),
        mesh=mesh,
        scratch_types=[
            pltpu.VMEM((per_w,), jnp.int32),
            pltpu.VMEM((per_w,), jnp.int32),
            pltpu.VMEM((W, D), jnp.float32),
            pltpu.VMEM((W, D), jnp.float32),
            pltpu.VMEM((W, D), jnp.float32),
            pltpu.VMEM((W, D), jnp.float32),
            pltpu.VMEM((LANES * LANES,), jnp.float32),
            pltpu.VMEM((per_w,), jnp.float32),
            pltpu.SemaphoreType.DMA,
            pltpu.SemaphoreType.DMA,
            pltpu.SemaphoreType.DMA,
            pltpu.SemaphoreType.DMA,
        ],
        compiler_params=cp,
    )


@jax.jit
def kernel(ufeat, ifeat, edge_index):
    E = edge_index.shape[1]
    src = edge_index[0].astype(jnp.int32)
    dst = edge_index[1].astype(jnp.int32)
    sr = _build_sc_call(E, 80)(ufeat, ifeat, src, dst)
    return sr.reshape(E, 1)
